# baseline (speedup 1.0000x reference)
import dataclasses
import functools

import jax
import jax.numpy as jnp
from jax import lax
from jax.experimental import pallas as pl
from jax.experimental.pallas import tpu as pltpu
from jax.experimental.pallas import tpu_sc as plsc

NC = 2
NS = 16
NW = NC * NS
LANES = 16
CHUNK = 128


def _sc_gather(ids, tok_table, *, seq_len, n_dim):
    rows_per_w = seq_len // NW
    vals_per_w = rows_per_w * n_dim
    n_chunks = rows_per_w // CHUNK
    n_groups = rows_per_w // LANES
    mesh = plsc.VectorSubcoreMesh(core_axis_name="c", subcore_axis_name="s")

    cp = pltpu.CompilerParams(use_tc_tiling_on_sc=False)
    if "needs_layout_passes" in pltpu.CompilerParams.__dataclass_fields__:
        cp = dataclasses.replace(cp, needs_layout_passes=False)

    @functools.partial(
        pl.kernel,
        mesh=mesh,
        compiler_params=cp,
        out_type=jax.ShapeDtypeStruct((seq_len * n_dim,), jnp.float32),
        scratch_types=[
            pltpu.VMEM((rows_per_w,), jnp.int32),
            pltpu.VMEM((n_chunks, CHUNK), jnp.int32),
            pltpu.VMEM((rows_per_w, 2 * n_dim), jnp.float32),
            pltpu.VMEM((vals_per_w,), jnp.float32),
            pltpu.SemaphoreType.DMA,
        ],
    )
    def k(ids_hbm, tok_hbm, out_hbm, idx_v, pidx_v, pair_v, out_v, gsem):
        wid = lax.axis_index("s") * NC + lax.axis_index("c")
        base = wid * rows_per_w

        pltpu.sync_copy(ids_hbm.at[pl.ds(base, rows_per_w)], idx_v)

        for c in range(n_chunks):
            @pl.loop(0, CHUNK // LANES)
            def _(j):
                vec = idx_v[pl.ds(c * CHUNK + j * LANES, LANES)]
                pidx_v[c, pl.ds(j * LANES, LANES)] = (
                    lax.shift_right_logical(vec, 1))

        gathers = []
        for c in range(n_chunks):
            gathers.append(pltpu.async_copy(
                tok_hbm.at[pidx_v.at[c]],
                pair_v.at[pl.ds(c * CHUNK, CHUNK)],
                gsem))
        for cp_ in gathers:
            cp_.wait()

        lane = jnp.arange(LANES, dtype=jnp.int32)

        @pl.loop(0, n_groups)
        def _(g):
            rvec = g * LANES + lane
            vec = idx_v[pl.ds(g * LANES, LANES)]
            colbase = (vec & 1) * n_dim
            flatbase = rvec * n_dim
            for j in range(n_dim):
                val = plsc.load_gather(pair_v, [rvec, colbase + j])
                plsc.store_scatter(out_v, [flatbase + j], val)

        pltpu.sync_copy(out_v, out_hbm.at[pl.ds(base * n_dim, vals_per_w)])

    return k(ids, tok_table)


def _tc_add(a, b):
    def body(a_ref, b_ref, o_ref):
        o_ref[...] = a_ref[...] + b_ref[...]

    return pl.pallas_call(
        body,
        out_shape=jax.ShapeDtypeStruct(a.shape, a.dtype),
    )(a, b)


def kernel(ids, tok_table, pos_table):
    seq_len = ids.shape[0]
    n_dim = tok_table.shape[1]
    tok_pairs = tok_table.reshape(tok_table.shape[0] // 2, 2 * n_dim)
    flat = _sc_gather(ids.astype(jnp.int32), tok_pairs,
                      seq_len=seq_len, n_dim=n_dim)
    gathered = flat.reshape(seq_len, n_dim)
    out = _tc_add(gathered, pos_table)
    return out[None]


# speedup vs baseline: 1.7794x; 1.7794x over previous
import dataclasses
import functools

import jax
import jax.numpy as jnp
from jax import lax
from jax.experimental import pallas as pl
from jax.experimental.pallas import tpu as pltpu
from jax.experimental.pallas import tpu_sc as plsc

NC = 2
NS = 16
NW = NC * NS
LANES = 16


def _sc_gather(ids, tok_table, *, seq_len, n_dim):
    rows_per_w = seq_len // NW
    n_groups = rows_per_w // LANES
    mesh = plsc.VectorSubcoreMesh(core_axis_name="c", subcore_axis_name="s")

    cp = pltpu.CompilerParams()
    if "needs_layout_passes" in pltpu.CompilerParams.__dataclass_fields__:
        cp = dataclasses.replace(cp, needs_layout_passes=False)

    @functools.partial(
        pl.kernel,
        mesh=mesh,
        compiler_params=cp,
        out_type=jax.ShapeDtypeStruct((seq_len, n_dim), jnp.float32),
        scratch_types=[
            pltpu.VMEM((rows_per_w,), jnp.int32),
            pltpu.VMEM((rows_per_w, n_dim), jnp.float32),
            pltpu.SemaphoreType.DMA,
        ],
    )
    def k(ids_hbm, tok_hbm, out_hbm, idx_v, rows_v, gsem):
        wid = lax.axis_index("s") * NC + lax.axis_index("c")
        base = wid * rows_per_w

        pltpu.sync_copy(ids_hbm.at[pl.ds(base, rows_per_w)], idx_v)

        lane = jnp.arange(LANES, dtype=jnp.int32)

        @pl.loop(0, n_groups)
        def _(g):
            vec = idx_v[pl.ds(g * LANES, LANES)]
            for l in range(LANES):
                i = jnp.sum(jnp.where(lane == l, vec, 0))
                pltpu.async_copy(
                    tok_hbm.at[pl.ds(i, 1)],
                    rows_v.at[pl.ds(g * LANES + l, 1)],
                    gsem)

        pltpu.make_async_copy(
            tok_hbm.at[pl.ds(0, rows_per_w)], rows_v, gsem).wait()

        pltpu.sync_copy(rows_v, out_hbm.at[pl.ds(base, rows_per_w)])

    return k(ids, tok_table)


def _tc_add(a, b):
    def body(a_ref, b_ref, o_ref):
        o_ref[...] = a_ref[...] + b_ref[...]

    return pl.pallas_call(
        body,
        out_shape=jax.ShapeDtypeStruct(a.shape, a.dtype),
    )(a, b)


def kernel(ids, tok_table, pos_table):
    seq_len = ids.shape[0]
    n_dim = tok_table.shape[1]
    gathered = _sc_gather(ids.astype(jnp.int32), tok_table,
                          seq_len=seq_len, n_dim=n_dim)
    out = _tc_add(gathered, pos_table)
    return out[None]
